# baseline (speedup 1.0000x reference)
import functools

import jax
import jax.numpy as jnp
from jax import lax
from jax.experimental import pallas as pl
from jax.experimental.pallas import tpu as pltpu


def _fused_kernel(xq0, xq1, xq2, xq3, w_ref, gamma_ref, beta_ref, o_ref,
                  xbf, gacc, sacc, ws_s, shift_s, *, na, bsz, nq, inv_m, eps):
    i = pl.program_id(0)
    quarters = (xq0, xq1, xq2, xq3)

    @pl.when(i < na)
    def _phase_a():
        xs = [q[0] for q in quarters]
        gram = lax.dot_general(xs[0], xs[0], (((1,), (1,)), ((), ())),
                               preferred_element_type=jnp.float32)
        for xm in xs[1:]:
            gram = gram + lax.dot_general(xm, xm, (((1,), (1,)), ((), ())),
                                          preferred_element_type=jnp.float32)
        s01 = xs[0].astype(jnp.float32) + xs[1].astype(jnp.float32)
        s23 = xs[2].astype(jnp.float32) + xs[3].astype(jnp.float32)
        ssum = jnp.sum(s01 + s23, axis=-1, keepdims=True)
        for k, xm in enumerate(xs):
            xbf[k * nq + i] = xm

        @pl.when(i == 0)
        def _():
            gacc[...] = gram
            sacc[...] = ssum

        @pl.when(i > 0)
        def _():
            gacc[...] = gacc[...] + gram
            sacc[...] = sacc[...] + ssum

    @pl.when(i >= na)
    def _phase_b():
        @pl.when(i == na)
        def _glue():
            w = w_ref[...]
            mean = jnp.dot(w, sacc[...],
                           preferred_element_type=jnp.float32) * inv_m
            wg = jnp.dot(w, gacc[...], preferred_element_type=jnp.float32)
            sumsq = jnp.sum(wg * w, axis=-1, keepdims=True)
            var = jnp.maximum(sumsq * inv_m - mean * mean, 0.0)
            scale = gamma_ref[...] * lax.rsqrt(var + eps)
            shift_s[...] = beta_ref[...] - mean * scale
            ws_s[...] = (w * scale).astype(jnp.bfloat16)

        ws = ws_s[...]
        sh = shift_s[...]
        for j in range(bsz):
            xb = xbf[(i - na) * bsz + j]
            y = jnp.dot(ws, xb, preferred_element_type=jnp.float32) + sh
            o_ref[j] = jnp.maximum(y, 0.0).astype(jnp.bfloat16)


def kernel(x, w, b, gamma, beta):
    del b
    eps = 1e-5
    N, Cin, D, H, W = x.shape
    Cout = w.shape[0]
    S = D * H * W
    M = N * S
    xr = x.reshape(N, Cin, S)

    NQ = N // 4
    xqs = [xr[k * NQ:(k + 1) * NQ].astype(jnp.bfloat16) for k in range(4)]

    B = 2 if N % 2 == 0 else 1
    NB_B = N // B
    NA = NQ

    body = functools.partial(_fused_kernel, na=NA, bsz=B, nq=NQ,
                             inv_m=1.0 / M, eps=eps)

    def _qmap():
        return lambda i: (jnp.minimum(i, NQ - 1), 0, 0)

    outb = pl.pallas_call(
        body,
        grid=(NA + NB_B,),
        in_specs=[pl.BlockSpec((1, Cin, S), _qmap()) for _ in range(4)] + [
            pl.BlockSpec((Cout, Cin), lambda i: (0, 0)),
            pl.BlockSpec((Cout, 1), lambda i: (0, 0)),
            pl.BlockSpec((Cout, 1), lambda i: (0, 0))],
        out_specs=pl.BlockSpec((B, Cout, S),
                               lambda i: (jnp.maximum(i - NA, 0), 0, 0)),
        out_shape=jax.ShapeDtypeStruct((N, Cout, S), jnp.bfloat16),
        scratch_shapes=[pltpu.VMEM((N, Cin, S), jnp.bfloat16),
                        pltpu.VMEM((Cin, Cin), jnp.float32),
                        pltpu.VMEM((Cin, 1), jnp.float32),
                        pltpu.VMEM((Cout, Cin), jnp.bfloat16),
                        pltpu.VMEM((Cout, 1), jnp.float32)],
        compiler_params=pltpu.CompilerParams(
            dimension_semantics=("arbitrary",),
            vmem_limit_bytes=38 << 20),
    )(*xqs, w, gamma.reshape(Cout, 1), beta.reshape(Cout, 1))

    return outb.astype(jnp.float32).reshape(N, Cout, D, H, W)


# speedup vs baseline: 1.1044x; 1.1044x over previous
import functools

import jax
import jax.numpy as jnp
from jax import lax
from jax.experimental import pallas as pl
from jax.experimental.pallas import tpu as pltpu


def _fused_kernel(x_ref, w_ref, gamma_ref, beta_ref, o_ref,
                  xbf, gacc, sacc, ws_s, shift_s, *, na, bsz, inv_m, eps):
    i = pl.program_id(0)

    @pl.when(i < na)
    def _phase_a():
        x0 = x_ref[0]
        gram = lax.dot_general(x0, x0, (((1,), (1,)), ((), ())),
                               preferred_element_type=jnp.float32)
        xacc = x0.astype(jnp.float32)
        xbf[i * bsz] = x0
        for j in range(1, bsz):
            xj = x_ref[j]
            gram = gram + lax.dot_general(xj, xj, (((1,), (1,)), ((), ())),
                                          preferred_element_type=jnp.float32)
            xacc = xacc + xj.astype(jnp.float32)
            xbf[i * bsz + j] = xj
        ssum = jnp.sum(xacc, axis=-1, keepdims=True)

        @pl.when(i == 0)
        def _():
            gacc[...] = gram
            sacc[...] = ssum

        @pl.when(i > 0)
        def _():
            gacc[...] = gacc[...] + gram
            sacc[...] = sacc[...] + ssum

    @pl.when(i >= na)
    def _phase_b():
        @pl.when(i == na)
        def _glue():
            w = w_ref[...]
            mean = jnp.dot(w, sacc[...],
                           preferred_element_type=jnp.float32) * inv_m
            wg = jnp.dot(w, gacc[...], preferred_element_type=jnp.float32)
            sumsq = jnp.sum(wg * w, axis=-1, keepdims=True)
            var = jnp.maximum(sumsq * inv_m - mean * mean, 0.0)
            scale = gamma_ref[...] * lax.rsqrt(var + eps)
            shift_s[...] = beta_ref[...] - mean * scale
            ws_s[...] = (w * scale).astype(jnp.bfloat16)

        ws = ws_s[...]
        sh = shift_s[...]
        for j in range(bsz):
            xb = xbf[(i - na) * bsz + j]
            y = jnp.dot(ws, xb, preferred_element_type=jnp.float32) + sh
            o_ref[j] = jnp.maximum(y, 0.0).astype(jnp.bfloat16)


def kernel(x, w, b, gamma, beta):
    del b
    eps = 1e-5
    N, Cin, D, H, W = x.shape
    Cout = w.shape[0]
    S = D * H * W
    M = N * S

    xbf_in = x.reshape(N, Cin, S).astype(jnp.bfloat16)

    B = 2 if N % 2 == 0 else 1
    NA = N // B

    body = functools.partial(_fused_kernel, na=NA, bsz=B, inv_m=1.0 / M,
                             eps=eps)
    outb = pl.pallas_call(
        body,
        grid=(2 * NA,),
        in_specs=[pl.BlockSpec((B, Cin, S),
                               lambda i: (jnp.minimum(i, NA - 1), 0, 0)),
                  pl.BlockSpec((Cout, Cin), lambda i: (0, 0)),
                  pl.BlockSpec((Cout, 1), lambda i: (0, 0)),
                  pl.BlockSpec((Cout, 1), lambda i: (0, 0))],
        out_specs=pl.BlockSpec((B, Cout, S),
                               lambda i: (jnp.maximum(i - NA, 0), 0, 0)),
        out_shape=jax.ShapeDtypeStruct((N, Cout, S), jnp.bfloat16),
        scratch_shapes=[pltpu.VMEM((N, Cin, S), jnp.bfloat16),
                        pltpu.VMEM((Cin, Cin), jnp.float32),
                        pltpu.VMEM((Cin, 1), jnp.float32),
                        pltpu.VMEM((Cout, Cin), jnp.bfloat16),
                        pltpu.VMEM((Cout, 1), jnp.float32)],
        compiler_params=pltpu.CompilerParams(
            dimension_semantics=("arbitrary",),
            vmem_limit_bytes=36 << 20),
    )(xbf_in, w, gamma.reshape(Cout, 1), beta.reshape(Cout, 1))

    return outb.astype(jnp.float32).reshape(N, Cout, D, H, W)


# speedup vs baseline: 1.1977x; 1.0844x over previous
import functools

import jax
import jax.numpy as jnp
from jax import lax
from jax.experimental import pallas as pl
from jax.experimental.pallas import tpu as pltpu


def _fused_kernel(xq0, xq1, xq2, xq3, w_ref, gamma_ref, beta_ref, o_ref,
                  xbf, gacc, sacc, ws_s, shift_s, *, na, nb_steps, bsz,
                  nq, inv_m, eps):
    i = pl.program_id(0)
    quarters = (xq0, xq1, xq2, xq3)

    @pl.when(i < na)
    def _phase_a():
        xs = [q[0] for q in quarters]
        gram = lax.dot_general(xs[0], xs[0], (((1,), (1,)), ((), ())),
                               preferred_element_type=jnp.float32)
        for xm in xs[1:]:
            gram = gram + lax.dot_general(xm, xm, (((1,), (1,)), ((), ())),
                                          preferred_element_type=jnp.float32)
        ssum = jnp.sum(xs[0] + xs[1] + xs[2] + xs[3], axis=-1, keepdims=True)
        for k, xm in enumerate(xs):
            xbf[k * nq + i] = xm.astype(jnp.bfloat16)

        @pl.when(i == 0)
        def _():
            gacc[...] = gram
            sacc[...] = ssum

        @pl.when(i > 0)
        def _():
            gacc[...] = gacc[...] + gram
            sacc[...] = sacc[...] + ssum

    @pl.when(i >= na)
    def _phase_b():
        @pl.when(i == na)
        def _glue():
            w = w_ref[...]
            mean = jnp.dot(w, sacc[...],
                           preferred_element_type=jnp.float32) * inv_m
            wg = jnp.dot(w, gacc[...], preferred_element_type=jnp.float32)
            sumsq = jnp.sum(wg * w, axis=-1, keepdims=True)
            var = jnp.maximum(sumsq * inv_m - mean * mean, 0.0)
            scale = gamma_ref[...] * lax.rsqrt(var + eps)
            shift_s[...] = beta_ref[...] - mean * scale
            ws_s[...] = (w * scale).astype(jnp.bfloat16)

        ws = ws_s[...]
        sh = shift_s[...]
        for j in range(bsz):
            xb = xbf[(i - na) * bsz + j]
            y = jnp.dot(ws, xb, preferred_element_type=jnp.float32) + sh
            o_ref[j] = jnp.maximum(y, 0.0).astype(jnp.bfloat16)


def kernel(x, w, b, gamma, beta):
    del b
    eps = 1e-5
    N, Cin, D, H, W = x.shape
    Cout = w.shape[0]
    S = D * H * W
    M = N * S
    xr = x.reshape(N, Cin, S)

    NQ = N // 4
    NA = NQ
    B = 2 if N % 2 == 0 else 1
    NB_B = N // B
    grid = (NA + NB_B,)

    body = functools.partial(_fused_kernel, na=NA, nb_steps=NB_B, bsz=B,
                             nq=NQ, inv_m=1.0 / M, eps=eps)

    def _qmap(k):
        return lambda i: (k * NQ + jnp.minimum(i, NQ - 1), 0, 0)

    outb = pl.pallas_call(
        body,
        grid=grid,
        in_specs=[pl.BlockSpec((1, Cin, S), _qmap(k)) for k in range(4)] + [
            pl.BlockSpec((Cout, Cin), lambda i: (0, 0)),
            pl.BlockSpec((Cout, 1), lambda i: (0, 0)),
            pl.BlockSpec((Cout, 1), lambda i: (0, 0))],
        out_specs=pl.BlockSpec((B, Cout, S),
                               lambda i: (jnp.maximum(i - NA, 0), 0, 0)),
        out_shape=jax.ShapeDtypeStruct((N, Cout, S), jnp.bfloat16),
        scratch_shapes=[pltpu.VMEM((N, Cin, S), jnp.bfloat16),
                        pltpu.VMEM((Cin, Cin), jnp.float32),
                        pltpu.VMEM((Cin, 1), jnp.float32),
                        pltpu.VMEM((Cout, Cin), jnp.bfloat16),
                        pltpu.VMEM((Cout, 1), jnp.float32)],
        compiler_params=pltpu.CompilerParams(
            dimension_semantics=("arbitrary",),
            vmem_limit_bytes=46 << 20),
    )(xr, xr, xr, xr, w, gamma.reshape(Cout, 1), beta.reshape(Cout, 1))

    return outb.astype(jnp.float32).reshape(N, Cout, D, H, W)
